# baseline (speedup 1.0000x reference)
import functools

import jax
import jax.numpy as jnp
from jax import lax
from jax.experimental import pallas as pl
from jax.experimental.pallas import tpu as pltpu
from jax.experimental.pallas import tpu_sc as plsc


def _pos_embed_sc(row_embed, col_embed, *, b, h, w, d):
    info = plsc.get_sparse_core_info()
    nc, ns, lanes = info.num_cores, info.num_subcores, info.num_lanes
    nw = nc * ns
    dpw = d // nw
    row_elems = h * w
    blk = dpw * row_elems

    mesh = plsc.VectorSubcoreMesh(core_axis_name="c", subcore_axis_name="s")

    @functools.partial(
        pl.kernel,
        out_type=jax.ShapeDtypeStruct((b * d * row_elems,), jnp.float32),
        mesh=mesh,
        scratch_types=[
            pltpu.VMEM((h, d), jnp.float32),
            pltpu.VMEM((w, d), jnp.float32),
            pltpu.VMEM((blk,), jnp.float32),
        ],
        compiler_params=pltpu.CompilerParams(
            use_tc_tiling_on_sc=False, needs_layout_passes=False
        ),
    )
    def body(row_hbm, col_hbm, out_hbm, row_v, col_v, outbuf):
        wid = lax.axis_index("s") * nc + lax.axis_index("c")
        d0 = wid * dpw

        pltpu.sync_copy(row_hbm.at[pl.ds(0, h), :], row_v)
        pltpu.sync_copy(col_hbm.at[pl.ds(0, w), :], col_v)

        jcount = w // lanes
        for dd in range(dpw):
            dd_idx = jnp.full((lanes,), d0 + dd, jnp.int32)
            c_chunks = [
                plsc.load_gather(
                    col_v,
                    [lax.iota(jnp.int32, lanes) + jc * lanes, dd_idx],
                )
                for jc in range(jcount)
            ]

            def irow(i, _, dd=dd, dd_idx=dd_idx, c_chunks=c_chunks):
                r = plsc.load_gather(
                    row_v, [jnp.full((lanes,), i, jnp.int32), dd_idx]
                )
                base = dd * row_elems + i * w
                for jc, c in enumerate(c_chunks):
                    outbuf[pl.ds(base + jc * lanes, lanes)] = r + c
                return 0

            lax.fori_loop(0, h, irow, 0)

        for bi in range(b):
            pltpu.sync_copy(
                outbuf,
                out_hbm.at[pl.ds(bi * d * row_elems + d0 * row_elems, blk)],
            )

    return body(row_embed, col_embed)


def kernel(x, row_embed, col_embed):
    b = x.shape[0]
    h, w = x.shape[-2], x.shape[-1]
    d = row_embed.shape[1]
    out = _pos_embed_sc(row_embed, col_embed, b=b, h=h, w=w, d=d)
    return out.reshape(b, d, h, w)


# speedup vs baseline: 1.0048x; 1.0048x over previous
import functools

import jax
import jax.numpy as jnp
from jax import lax
from jax.experimental import pallas as pl
from jax.experimental.pallas import tpu as pltpu
from jax.experimental.pallas import tpu_sc as plsc


def _pos_embed_sc(row_embed, col_embed, *, b, h, w, d):
    info = plsc.get_sparse_core_info()
    nc, ns, lanes = info.num_cores, info.num_subcores, info.num_lanes
    nw = nc * ns
    dpw = d // nw

    mesh = plsc.VectorSubcoreMesh(core_axis_name="c", subcore_axis_name="s")

    @functools.partial(
        pl.kernel,
        out_type=jax.ShapeDtypeStruct((b, d, h, w), jnp.float32),
        mesh=mesh,
        scratch_types=[
            pltpu.VMEM((h, d), jnp.float32),
            pltpu.VMEM((w, d), jnp.float32),
            pltpu.VMEM((dpw, h, w), jnp.float32),
            pltpu.SemaphoreType.DMA,
        ],
        compiler_params=pltpu.CompilerParams(
            use_tc_tiling_on_sc=False, needs_layout_passes=False
        ),
    )
    def body(row_hbm, col_hbm, out_hbm, row_v, col_v, outbuf, sem):
        wid = lax.axis_index("s") * nc + lax.axis_index("c")
        d0 = wid * dpw

        pltpu.sync_copy(row_hbm.at[pl.ds(0, h), :], row_v)
        pltpu.sync_copy(col_hbm.at[pl.ds(0, w), :], col_v)

        jcount = w // lanes
        for dd in range(dpw):
            dd_idx = jnp.full((lanes,), d0 + dd, jnp.int32)
            c_chunks = [
                plsc.load_gather(
                    col_v,
                    [lax.iota(jnp.int32, lanes) + jc * lanes, dd_idx],
                )
                for jc in range(jcount)
            ]

            def irow(i, _, dd=dd, dd_idx=dd_idx, c_chunks=c_chunks):
                r = plsc.load_gather(
                    row_v, [jnp.full((lanes,), i, jnp.int32), dd_idx]
                )
                for jc, c in enumerate(c_chunks):
                    outbuf[dd, i, pl.ds(jc * lanes, lanes)] = r + c
                return 0

            lax.fori_loop(0, h, irow, 0)

        copies = [
            pltpu.async_copy(outbuf, out_hbm.at[bi, pl.ds(d0, dpw)], sem)
            for bi in range(b)
        ]
        for c in copies:
            c.wait()

    return body(row_embed, col_embed)


def kernel(x, row_embed, col_embed):
    b = x.shape[0]
    h, w = x.shape[-2], x.shape[-1]
    d = row_embed.shape[1]
    return _pos_embed_sc(row_embed, col_embed, b=b, h=h, w=w, d=d)


# speedup vs baseline: 1.0719x; 1.0669x over previous
import functools

import jax
import jax.numpy as jnp
from jax import lax
from jax.experimental import pallas as pl
from jax.experimental.pallas import tpu as pltpu
from jax.experimental.pallas import tpu_sc as plsc


def _pos_embed_sc(row_embed, col_embed, *, b, h, w, d):
    info = plsc.get_sparse_core_info()
    nc, ns, lanes = info.num_cores, info.num_subcores, info.num_lanes
    nw = nc * ns
    dpw = d // nw

    mesh = plsc.VectorSubcoreMesh(core_axis_name="c", subcore_axis_name="s")

    @functools.partial(
        pl.kernel,
        out_type=jax.ShapeDtypeStruct((b, d, h, w), jnp.float32),
        mesh=mesh,
        scratch_types=[
            pltpu.VMEM((h, d), jnp.float32),
            pltpu.VMEM((w, d), jnp.float32),
            pltpu.VMEM((dpw, h, w), jnp.float32),
            pltpu.SemaphoreType.DMA,
        ],
        compiler_params=pltpu.CompilerParams(
            use_tc_tiling_on_sc=True, needs_layout_passes=False
        ),
    )
    def body(row_hbm, col_hbm, out_hbm, row_v, col_v, outbuf, sem):
        wid = lax.axis_index("s") * nc + lax.axis_index("c")
        d0 = wid * dpw

        pltpu.sync_copy(row_hbm.at[pl.ds(0, h), :], row_v)
        pltpu.sync_copy(col_hbm.at[pl.ds(0, w), :], col_v)

        jcount = w // lanes
        for dd in range(dpw):
            dd_idx = jnp.full((lanes,), d0 + dd, jnp.int32)
            c_chunks = [
                plsc.load_gather(
                    col_v,
                    [lax.iota(jnp.int32, lanes) + jc * lanes, dd_idx],
                )
                for jc in range(jcount)
            ]

            def irow(i, _, dd=dd, dd_idx=dd_idx, c_chunks=c_chunks):
                r = plsc.load_gather(
                    row_v, [jnp.full((lanes,), i, jnp.int32), dd_idx]
                )
                for jc, c in enumerate(c_chunks):
                    outbuf[dd, i, pl.ds(jc * lanes, lanes)] = r + c
                return 0

            lax.fori_loop(0, h, irow, 0)

        copies = [
            pltpu.async_copy(outbuf, out_hbm.at[bi, pl.ds(d0, dpw)], sem)
            for bi in range(b)
        ]
        for c in copies:
            c.wait()

    return body(row_embed, col_embed)


def kernel(x, row_embed, col_embed):
    b = x.shape[0]
    h, w = x.shape[-2], x.shape[-1]
    d = row_embed.shape[1]
    return _pos_embed_sc(row_embed, col_embed, b=b, h=h, w=w, d=d)


# speedup vs baseline: 2.2034x; 2.0556x over previous
import functools

import jax
import jax.numpy as jnp
from jax import lax
from jax.experimental import pallas as pl
from jax.experimental.pallas import tpu as pltpu
from jax.experimental.pallas import tpu_sc as plsc


def _pos_embed_sc(row_embed, col_embed, *, b, h, w, d):
    info = plsc.get_sparse_core_info()
    nc, ns, lanes = info.num_cores, info.num_subcores, info.num_lanes
    nw = nc * ns
    dchunks = d // lanes

    mesh = plsc.VectorSubcoreMesh(core_axis_name="c", subcore_axis_name="s")

    @functools.partial(
        pl.kernel,
        out_type=jax.ShapeDtypeStruct((b, h, w, d), jnp.float32),
        mesh=mesh,
        scratch_types=[
            pltpu.VMEM((h, d), jnp.float32),
            pltpu.VMEM((w, d), jnp.float32),
            pltpu.VMEM((w, d), jnp.float32),
            pltpu.SemaphoreType.DMA,
        ],
        compiler_params=pltpu.CompilerParams(
            use_tc_tiling_on_sc=True, needs_layout_passes=False
        ),
    )
    def body(row_hbm, col_hbm, out_hbm, row_v, col_v, blk_v, sem):
        i = lax.axis_index("s") * nc + lax.axis_index("c")

        pltpu.sync_copy(row_hbm.at[pl.ds(0, h), :], row_v)
        pltpu.sync_copy(col_hbm.at[pl.ds(0, w), :], col_v)

        r_chunks = [row_v[i, pl.ds(k * lanes, lanes)] for k in range(dchunks)]
        for j in range(w):
            for k in range(dchunks):
                sl = pl.ds(k * lanes, lanes)
                blk_v[j, sl] = col_v[j, sl] + r_chunks[k]

        copies = [
            pltpu.async_copy(blk_v, out_hbm.at[bi, i], sem) for bi in range(b)
        ]
        for c in copies:
            c.wait()

    return body(row_embed, col_embed)


def kernel(x, row_embed, col_embed):
    b = x.shape[0]
    h, w = x.shape[-2], x.shape[-1]
    d = row_embed.shape[1]
    out_bhwd = _pos_embed_sc(row_embed, col_embed, b=b, h=h, w=w, d=d)
    return jnp.transpose(out_bhwd, (0, 3, 1, 2))


# speedup vs baseline: 2.3955x; 1.0872x over previous
import functools

import jax
import jax.numpy as jnp
from jax import lax
from jax.experimental import pallas as pl
from jax.experimental.pallas import tpu as pltpu
from jax.experimental.pallas import tpu_sc as plsc


def _pos_embed_sc(row_embed, col_embed, *, b, h, w, d):
    info = plsc.get_sparse_core_info()
    nc, ns, lanes = info.num_cores, info.num_subcores, info.num_lanes
    nw = nc * ns
    dchunks = d // lanes

    mesh = plsc.VectorSubcoreMesh(core_axis_name="c", subcore_axis_name="s")

    @functools.partial(
        pl.kernel,
        out_type=jax.ShapeDtypeStruct((b, h, w, d), jnp.float32),
        mesh=mesh,
        scratch_types=[
            pltpu.VMEM((8, d), jnp.float32),
            pltpu.VMEM((w, d), jnp.float32),
            pltpu.VMEM((w, d), jnp.float32),
            pltpu.SemaphoreType.DMA,
            pltpu.SemaphoreType.DMA,
        ],
        compiler_params=pltpu.CompilerParams(
            use_tc_tiling_on_sc=True, needs_layout_passes=False
        ),
    )
    def body(row_hbm, col_hbm, out_hbm, row_v, col_v, blk_v, stage_sem, sem):
        i = lax.axis_index("s") * nc + lax.axis_index("c")

        g0 = (i // 8) * 8
        st_r = pltpu.async_copy(
            row_hbm.at[pl.ds(g0, 8), :], row_v, stage_sem
        )
        st_c = pltpu.async_copy(col_hbm.at[pl.ds(0, w), :], col_v, stage_sem)
        st_r.wait()
        st_c.wait()

        ii = i - g0
        r_chunks = [row_v[ii, pl.ds(k * lanes, lanes)] for k in range(dchunks)]

        def jrow(j, _):
            for k in range(dchunks):
                sl = pl.ds(k * lanes, lanes)
                blk_v[j, sl] = col_v[j, sl] + r_chunks[k]
            return 0

        copies = []
        hw = w // 2
        for half in range(2):
            lax.fori_loop(half * hw, (half + 1) * hw, jrow, 0)
            src = blk_v.at[pl.ds(half * hw, hw), :]
            copies += [
                pltpu.async_copy(
                    src, out_hbm.at[bi, i, pl.ds(half * hw, hw)], sem
                )
                for bi in range(b)
            ]
        for c in copies:
            c.wait()

    return body(row_embed, col_embed)


def kernel(x, row_embed, col_embed):
    b = x.shape[0]
    h, w = x.shape[-2], x.shape[-1]
    d = row_embed.shape[1]
    out_bhwd = _pos_embed_sc(row_embed, col_embed, b=b, h=h, w=w, d=d)
    return jnp.transpose(out_bhwd, (0, 3, 1, 2))


# speedup vs baseline: 2.4403x; 1.0187x over previous
import functools

import jax
import jax.numpy as jnp
from jax import lax
from jax.experimental import pallas as pl
from jax.experimental.pallas import tpu as pltpu
from jax.experimental.pallas import tpu_sc as plsc


def _pos_embed_sc(row_embed, col_embed, *, b, h, w, d):
    info = plsc.get_sparse_core_info()
    nc, ns, lanes = info.num_cores, info.num_subcores, info.num_lanes
    nw = nc * ns
    dchunks = d // lanes

    mesh = plsc.VectorSubcoreMesh(core_axis_name="c", subcore_axis_name="s")

    @functools.partial(
        pl.kernel,
        out_type=jax.ShapeDtypeStruct((b, h, w, d), jnp.float32),
        mesh=mesh,
        scratch_types=[
            pltpu.VMEM((8, d), jnp.float32),
            pltpu.VMEM((w, d), jnp.float32),
            pltpu.VMEM((w, d), jnp.float32),
            pltpu.SemaphoreType.DMA,
            pltpu.SemaphoreType.DMA,
        ],
        compiler_params=pltpu.CompilerParams(
            use_tc_tiling_on_sc=True, needs_layout_passes=False
        ),
    )
    def body(row_hbm, col_hbm, out_hbm, row_v, col_v, blk_v, stage_sem, sem):
        i = lax.axis_index("s") * nc + lax.axis_index("c")

        g0 = (i // 8) * 8
        hw = w // 2
        st_r = pltpu.async_copy(
            row_hbm.at[pl.ds(g0, 8), :], row_v, stage_sem
        )
        st_c = [
            pltpu.async_copy(
                col_hbm.at[pl.ds(half * hw, hw), :],
                col_v.at[pl.ds(half * hw, hw), :],
                stage_sem,
            )
            for half in range(2)
        ]
        st_r.wait()

        ii = i - g0
        r_chunks = [row_v[ii, pl.ds(k * lanes, lanes)] for k in range(dchunks)]

        def jrow(j, _):
            for k in range(dchunks):
                sl = pl.ds(k * lanes, lanes)
                blk_v[j, sl] = col_v[j, sl] + r_chunks[k]
            return 0

        copies = []
        for half in range(2):
            st_c[half].wait()
            lax.fori_loop(half * hw, (half + 1) * hw, jrow, 0)
            src = blk_v.at[pl.ds(half * hw, hw), :]
            copies += [
                pltpu.async_copy(
                    src, out_hbm.at[bi, i, pl.ds(half * hw, hw)], sem
                )
                for bi in range(b)
            ]
        for c in copies:
            c.wait()

    return body(row_embed, col_embed)


def kernel(x, row_embed, col_embed):
    b = x.shape[0]
    h, w = x.shape[-2], x.shape[-1]
    d = row_embed.shape[1]
    out_bhwd = _pos_embed_sc(row_embed, col_embed, b=b, h=h, w=w, d=d)
    return jnp.transpose(out_bhwd, (0, 3, 1, 2))
